# baseline (speedup 1.0000x reference)
import jax
import jax.numpy as jnp
from jax.experimental import pallas as pl
from jax.experimental.pallas import tpu as pltpu

MAX_OUT = 100
IOU_THR = 0.5
SCORE_THR = 0.01
ROWS = 160
LANES = 128
NPAD = ROWS * LANES
NEG_INF = float("-inf")


def _decode_nms_kernel(
    lt_ref, db_ref, boxes_ref, cls_ref, sc_ref, num_ref,
    s_scr, x1_scr, y1_scr, x2_scr, y2_scr, ar_scr, cl_scr,
):
    ax1 = db_ref[0]
    ay1 = db_ref[1]
    ax2 = db_ref[2]
    ay2 = db_ref[3]
    acx = (ax2 + ax1) * 0.5
    acy = (ay2 + ay1) * 0.5
    aw = ax2 - ax1
    ah = ay2 - ay1

    pcx = lt_ref[0, 0] * aw + acx
    pcy = lt_ref[0, 1] * ah + acy
    pw = jnp.exp(lt_ref[0, 2]) * aw
    ph = jnp.exp(lt_ref[0, 3]) * ah
    x1 = jnp.clip(pcx - pw * 0.5, 0.0, 1.0)
    y1 = jnp.clip(pcy - ph * 0.5, 0.0, 1.0)
    x2 = jnp.clip(pcx + pw * 0.5, 0.0, 1.0)
    y2 = jnp.clip(pcy + ph * 0.5, 0.0, 1.0)
    x1_scr[:, :] = x1
    y1_scr[:, :] = y1
    x2_scr[:, :] = x2
    y2_scr[:, :] = y2
    ar_scr[:, :] = (x2 - x1) * (y2 - y1)

    m = lt_ref[0, 4]
    for c in range(5, 25):
        m = jnp.maximum(m, lt_ref[0, c])
    ssum = jnp.exp(lt_ref[0, 4] - m)
    best = lt_ref[0, 4]
    cls = jnp.zeros((ROWS, LANES), dtype=jnp.int32)
    for c in range(5, 25):
        lc = lt_ref[0, c]
        ssum = ssum + jnp.exp(lc - m)
        gt = lc > best
        best = jnp.where(gt, lc, best)
        cls = jnp.where(gt, jnp.int32(c - 4), cls)
    score = 1.0 / ssum
    cl_scr[:, :] = cls

    s0 = jnp.where(cls != 0, score, NEG_INF)
    s0 = jnp.where(s0 < SCORE_THR, NEG_INF, s0)
    s_scr[:, :] = s0

    iota = (
        jax.lax.broadcasted_iota(jnp.int32, (ROWS, LANES), 0) * LANES
        + jax.lax.broadcasted_iota(jnp.int32, (ROWS, LANES), 1)
    )
    lane1 = jax.lax.broadcasted_iota(jnp.int32, (1, LANES), 1)

    boxes_ref[...] = jnp.zeros_like(boxes_ref)
    cls_ref[...] = jnp.zeros_like(cls_ref)
    sc_ref[...] = jnp.zeros_like(sc_ref)

    def argmax_flat():
        s = s_scr[:, :]
        msc = jnp.max(s)
        idx = jnp.min(jnp.where(s == msc, iota, jnp.int32(NPAD)))
        return msc, idx

    z128 = jnp.zeros((1, LANES), dtype=jnp.float32)

    def cond(state):
        n, msc, _idx, _sx1, _sy1, _sx2, _sy2, _sar = state
        return (n < MAX_OUT) & (msc != NEG_INF)

    def body(state):
        n, _msc, idx, sx1, sy1, sx2, sy2, sar = state
        r = idx // LANES
        c = idx % LANES
        clane = lane1 == c

        def pick(ref):
            row = ref[pl.ds(r, 1), :]
            return jnp.max(jnp.where(clane, row, NEG_INF))

        bx1 = pick(x1_scr)
        by1 = pick(y1_scr)
        bx2 = pick(x2_scr)
        by2 = pick(y2_scr)
        barea = pick(ar_scr)

        xx1 = jnp.maximum(bx1, sx1)
        yy1 = jnp.maximum(by1, sy1)
        xx2 = jnp.minimum(bx2, sx2)
        yy2 = jnp.minimum(by2, sy2)
        inter = jnp.maximum(xx2 - xx1, 0.0) * jnp.maximum(yy2 - yy1, 0.0)
        iou = inter / (barea + sar - inter + 1e-9)
        suppressed = jnp.max(jnp.where((lane1 < n) & (iou > IOU_THR), 1, 0)) > 0

        srow = s_scr[pl.ds(r, 1), :]
        bsc = jnp.max(jnp.where(clane, srow, NEG_INF))
        s_scr[pl.ds(r, 1), :] = jnp.where(clane, NEG_INF, srow)

        keep = jnp.logical_not(suppressed)

        @pl.when(keep)
        def _():
            crow = cl_scr[pl.ds(r, 1), :]
            bcls = jnp.max(jnp.where(clane, crow, 0))
            li = jax.lax.broadcasted_iota(jnp.int32, (1, 4), 1)
            row4 = jnp.where(
                li == 0, bx1, jnp.where(li == 1, by1, jnp.where(li == 2, bx2, by2))
            )
            boxes_ref[0, pl.ds(n, 1), :] = row4
            cls_ref[0, pl.ds(n, 1), :] = jnp.reshape(bcls, (1, 1))
            sc_ref[0, pl.ds(n, 1), :] = jnp.reshape(bsc, (1, 1))

        nlane = lane1 == n
        sx1 = jnp.where(keep & nlane, bx1, sx1)
        sy1 = jnp.where(keep & nlane, by1, sy1)
        sx2 = jnp.where(keep & nlane, bx2, sx2)
        sy2 = jnp.where(keep & nlane, by2, sy2)
        sar = jnp.where(keep & nlane, barea, sar)
        n = n + keep.astype(jnp.int32)

        msc, idx = argmax_flat()
        return n, msc, idx, sx1, sy1, sx2, sy2, sar

    msc0, idx0 = argmax_flat()
    state = (jnp.int32(0), msc0, idx0, z128, z128, z128, z128, z128)
    state = jax.lax.while_loop(cond, body, state)
    num_ref[0] = jnp.reshape(state[0], (1, 1))


@jax.jit
def _run(lt, db):
    return pl.pallas_call(
        _decode_nms_kernel,
        grid=(4,),
        in_specs=[
            pl.BlockSpec((1, 25, ROWS, LANES), lambda b: (b, 0, 0, 0)),
            pl.BlockSpec((4, ROWS, LANES), lambda b: (0, 0, 0)),
        ],
        out_specs=[
            pl.BlockSpec((1, MAX_OUT, 4), lambda b: (b, 0, 0)),
            pl.BlockSpec((1, MAX_OUT, 1), lambda b: (b, 0, 0)),
            pl.BlockSpec((1, MAX_OUT, 1), lambda b: (b, 0, 0)),
            pl.BlockSpec((1, 1, 1), lambda b: (b, 0, 0)),
        ],
        out_shape=[
            jax.ShapeDtypeStruct((4, MAX_OUT, 4), jnp.float32),
            jax.ShapeDtypeStruct((4, MAX_OUT, 1), jnp.int32),
            jax.ShapeDtypeStruct((4, MAX_OUT, 1), jnp.float32),
            jax.ShapeDtypeStruct((4, 1, 1), jnp.int32),
        ],
        scratch_shapes=[
            pltpu.VMEM((ROWS, LANES), jnp.float32),
            pltpu.VMEM((ROWS, LANES), jnp.float32),
            pltpu.VMEM((ROWS, LANES), jnp.float32),
            pltpu.VMEM((ROWS, LANES), jnp.float32),
            pltpu.VMEM((ROWS, LANES), jnp.float32),
            pltpu.VMEM((ROWS, LANES), jnp.float32),
            pltpu.VMEM((ROWS, LANES), jnp.int32),
        ],
        compiler_params=pltpu.CompilerParams(
            dimension_semantics=("arbitrary",),
        ),
    )(lt, db)


def kernel(logits, default_boxes):
    b, n, c = logits.shape
    lt = jnp.transpose(logits, (0, 2, 1))
    lt = jnp.pad(lt, ((0, 0), (0, 0), (0, NPAD - n))).reshape(b, c, ROWS, LANES)
    db = jnp.pad(default_boxes.T, ((0, 0), (0, NPAD - n))).reshape(4, ROWS, LANES)
    det_boxes, det_cls, det_sc, det_num = _run(lt, db)
    return (
        det_boxes,
        det_cls[:, :, 0],
        det_sc[:, :, 0],
        det_num[:, 0, 0],
    )


# speedup vs baseline: 1.2157x; 1.2157x over previous
import jax
import jax.numpy as jnp
from jax.experimental import pallas as pl
from jax.experimental.pallas import tpu as pltpu

MAX_OUT = 100
IOU_THR = 0.5
SCORE_THR = 0.01
ROWS = 160
LANES = 128
NPAD = ROWS * LANES
NEG_INF = float("-inf")
B = 4


def _decode_nms_kernel(
    lt_ref, db_ref, boxes_ref, cls_ref, sc_ref, num_ref,
    s_scr, x1_scr, y1_scr, x2_scr, y2_scr, ar_scr, cl_scr, sc_scr,
):
    ax1 = db_ref[0]
    ay1 = db_ref[1]
    ax2 = db_ref[2]
    ay2 = db_ref[3]
    acx = (ax2 + ax1) * 0.5
    acy = (ay2 + ay1) * 0.5
    aw = ax2 - ax1
    ah = ay2 - ay1

    for b in range(B):
        pcx = lt_ref[b, 0] * aw + acx
        pcy = lt_ref[b, 1] * ah + acy
        pw = jnp.exp(lt_ref[b, 2]) * aw
        ph = jnp.exp(lt_ref[b, 3]) * ah
        x1 = jnp.clip(pcx - pw * 0.5, 0.0, 1.0)
        y1 = jnp.clip(pcy - ph * 0.5, 0.0, 1.0)
        x2 = jnp.clip(pcx + pw * 0.5, 0.0, 1.0)
        y2 = jnp.clip(pcy + ph * 0.5, 0.0, 1.0)
        x1_scr[b] = x1
        y1_scr[b] = y1
        x2_scr[b] = x2
        y2_scr[b] = y2
        ar_scr[b] = (x2 - x1) * (y2 - y1)

        m = lt_ref[b, 4]
        for c in range(5, 25):
            m = jnp.maximum(m, lt_ref[b, c])
        ssum = jnp.exp(lt_ref[b, 4] - m)
        best = lt_ref[b, 4]
        cls = jnp.zeros((ROWS, LANES), dtype=jnp.int32)
        for c in range(5, 25):
            lc = lt_ref[b, c]
            ssum = ssum + jnp.exp(lc - m)
            gt = lc > best
            best = jnp.where(gt, lc, best)
            cls = jnp.where(gt, jnp.int32(c - 4), cls)
        score = 1.0 / ssum
        cl_scr[b] = cls
        sc_scr[b] = score
        s0 = jnp.where(cls != 0, score, NEG_INF)
        s0 = jnp.where(s0 < SCORE_THR, NEG_INF, s0)
        s_scr[b] = s0

    iota = (
        jax.lax.broadcasted_iota(jnp.int32, (ROWS, LANES), 0) * LANES
        + jax.lax.broadcasted_iota(jnp.int32, (ROWS, LANES), 1)
    )
    lane1 = jax.lax.broadcasted_iota(jnp.int32, (1, LANES), 1)
    li4 = jax.lax.broadcasted_iota(jnp.int32, (1, 4), 1)

    boxes_ref[...] = jnp.zeros_like(boxes_ref)
    cls_ref[...] = jnp.zeros_like(cls_ref)
    sc_ref[...] = jnp.zeros_like(sc_ref)

    def body(t, ns):
        new_ns = []
        for b in range(B):
            n = ns[b]
            s = s_scr[b]
            msc = jnp.max(s)
            valid = msc != NEG_INF
            idx = jnp.min(jnp.where(s == msc, iota, jnp.int32(NPAD)))
            r = idx // LANES
            c = idx % LANES
            clane = lane1 == c

            def pick(ref, b=b, r=r, clane=clane):
                row = ref[b, pl.ds(r, 1), :]
                return jnp.max(jnp.where(clane, row, NEG_INF))

            bx1 = pick(x1_scr)
            by1 = pick(y1_scr)
            bx2 = pick(x2_scr)
            by2 = pick(y2_scr)
            barea = (bx2 - bx1) * (by2 - by1)

            xx1 = jnp.maximum(bx1, x1_scr[b])
            yy1 = jnp.maximum(by1, y1_scr[b])
            xx2 = jnp.minimum(bx2, x2_scr[b])
            yy2 = jnp.minimum(by2, y2_scr[b])
            inter = jnp.maximum(xx2 - xx1, 0.0) * jnp.maximum(yy2 - yy1, 0.0)
            iou = inter / (barea + ar_scr[b] - inter + 1e-9)
            supp = (iou > IOU_THR) | (iota == idx)

            @pl.when(valid)
            def _(b=b, supp=supp, s=s, r=r, clane=clane, t=t,
                  bx1=bx1, by1=by1, bx2=bx2, by2=by2):
                s_scr[b] = jnp.where(supp, NEG_INF, s)
                scrow = sc_scr[b, pl.ds(r, 1), :]
                bsc = jnp.max(jnp.where(clane, scrow, NEG_INF))
                crow = cl_scr[b, pl.ds(r, 1), :]
                bcls = jnp.max(jnp.where(clane, crow, 0))
                row4 = jnp.where(
                    li4 == 0, bx1,
                    jnp.where(li4 == 1, by1, jnp.where(li4 == 2, bx2, by2)),
                )
                boxes_ref[b, pl.ds(t, 1), :] = row4
                cls_ref[b, pl.ds(t, 1), :] = jnp.reshape(bcls, (1, 1))
                sc_ref[b, pl.ds(t, 1), :] = jnp.reshape(bsc, (1, 1))

            new_ns.append(n + valid.astype(jnp.int32))
        return tuple(new_ns)

    ns = jax.lax.fori_loop(
        0, MAX_OUT, body, tuple(jnp.int32(0) for _ in range(B))
    )
    for b in range(B):
        num_ref[b] = jnp.reshape(ns[b], (1, 1))


@jax.jit
def _run(lt, db):
    return pl.pallas_call(
        _decode_nms_kernel,
        in_specs=[
            pl.BlockSpec((B, 25, ROWS, LANES), lambda: (0, 0, 0, 0)),
            pl.BlockSpec((4, ROWS, LANES), lambda: (0, 0, 0)),
        ],
        out_specs=[
            pl.BlockSpec((B, MAX_OUT, 4), lambda: (0, 0, 0)),
            pl.BlockSpec((B, MAX_OUT, 1), lambda: (0, 0, 0)),
            pl.BlockSpec((B, MAX_OUT, 1), lambda: (0, 0, 0)),
            pl.BlockSpec((B, 1, 1), lambda: (0, 0, 0)),
        ],
        out_shape=[
            jax.ShapeDtypeStruct((B, MAX_OUT, 4), jnp.float32),
            jax.ShapeDtypeStruct((B, MAX_OUT, 1), jnp.int32),
            jax.ShapeDtypeStruct((B, MAX_OUT, 1), jnp.float32),
            jax.ShapeDtypeStruct((B, 1, 1), jnp.int32),
        ],
        scratch_shapes=[
            pltpu.VMEM((B, ROWS, LANES), jnp.float32),
            pltpu.VMEM((B, ROWS, LANES), jnp.float32),
            pltpu.VMEM((B, ROWS, LANES), jnp.float32),
            pltpu.VMEM((B, ROWS, LANES), jnp.float32),
            pltpu.VMEM((B, ROWS, LANES), jnp.float32),
            pltpu.VMEM((B, ROWS, LANES), jnp.float32),
            pltpu.VMEM((B, ROWS, LANES), jnp.int32),
            pltpu.VMEM((B, ROWS, LANES), jnp.float32),
        ],
    )(lt, db)


def kernel(logits, default_boxes):
    b, n, c = logits.shape
    lt = jnp.transpose(logits, (0, 2, 1))
    lt = jnp.pad(lt, ((0, 0), (0, 0), (0, NPAD - n))).reshape(b, c, ROWS, LANES)
    db = jnp.pad(default_boxes.T, ((0, 0), (0, NPAD - n))).reshape(4, ROWS, LANES)
    det_boxes, det_cls, det_sc, det_num = _run(lt, db)
    return (
        det_boxes,
        det_cls[:, :, 0],
        det_sc[:, :, 0],
        det_num[:, 0, 0],
    )


# speedup vs baseline: 4.6116x; 3.7934x over previous
import functools

import jax
import jax.numpy as jnp
from jax import lax
from jax.experimental import pallas as pl
from jax.experimental.pallas import tpu as pltpu
from jax.experimental.pallas import tpu_sc as plsc

MAX_OUT = 100
IOU_THR = 0.5
SCORE_THR = 0.01
ROWS = 160
LANES = 128
NPAD = ROWS * LANES
NEG_INF = float("-inf")

L = 16
NCHUNK = NPAD // L
NM1V = NCHUNK // L
NM2V = NM1V // L
SELPAD = 112


def _prep_kernel(lt_ref, db_ref, f_ref, c_ref):
    ax1 = db_ref[0]
    ay1 = db_ref[1]
    ax2 = db_ref[2]
    ay2 = db_ref[3]
    acx = (ax2 + ax1) * 0.5
    acy = (ay2 + ay1) * 0.5
    aw = ax2 - ax1
    ah = ay2 - ay1

    pcx = lt_ref[0, 0] * aw + acx
    pcy = lt_ref[0, 1] * ah + acy
    pw = jnp.exp(lt_ref[0, 2]) * aw
    ph = jnp.exp(lt_ref[0, 3]) * ah
    f_ref[0, 1] = jnp.clip(pcx - pw * 0.5, 0.0, 1.0)
    f_ref[0, 2] = jnp.clip(pcy - ph * 0.5, 0.0, 1.0)
    f_ref[0, 3] = jnp.clip(pcx + pw * 0.5, 0.0, 1.0)
    f_ref[0, 4] = jnp.clip(pcy + ph * 0.5, 0.0, 1.0)

    m = lt_ref[0, 4]
    for c in range(5, 25):
        m = jnp.maximum(m, lt_ref[0, c])
    ssum = jnp.exp(lt_ref[0, 4] - m)
    best = lt_ref[0, 4]
    cls = jnp.zeros((ROWS, LANES), dtype=jnp.int32)
    for c in range(5, 25):
        lc = lt_ref[0, c]
        ssum = ssum + jnp.exp(lc - m)
        gt = lc > best
        best = jnp.where(gt, lc, best)
        cls = jnp.where(gt, jnp.int32(c - 4), cls)
    score = 1.0 / ssum
    c_ref[0] = cls
    s0 = jnp.where(cls != 0, score, NEG_INF)
    s0 = jnp.where(s0 < SCORE_THR, NEG_INF, s0)
    f_ref[0, 0] = s0


@jax.jit
def _prep(lt, db):
    return pl.pallas_call(
        _prep_kernel,
        grid=(4,),
        in_specs=[
            pl.BlockSpec((1, 25, ROWS, LANES), lambda b: (b, 0, 0, 0)),
            pl.BlockSpec((4, ROWS, LANES), lambda b: (0, 0, 0)),
        ],
        out_specs=[
            pl.BlockSpec((1, 5, ROWS, LANES), lambda b: (b, 0, 0, 0)),
            pl.BlockSpec((1, ROWS, LANES), lambda b: (b, 0, 0)),
        ],
        out_shape=[
            jax.ShapeDtypeStruct((4, 5, ROWS, LANES), jnp.float32),
            jax.ShapeDtypeStruct((4, ROWS, LANES), jnp.int32),
        ],
        compiler_params=pltpu.CompilerParams(
            dimension_semantics=("arbitrary",),
        ),
    )(lt, db)


def _sc_nms(f2, c2):
    mesh = plsc.VectorSubcoreMesh(core_axis_name="c", subcore_axis_name="s")

    @functools.partial(
        pl.kernel,
        mesh=mesh,
        out_type=[
            jax.ShapeDtypeStruct((4 * SELPAD,), jnp.float32),
            jax.ShapeDtypeStruct((4 * SELPAD,), jnp.float32),
            jax.ShapeDtypeStruct((4 * SELPAD,), jnp.float32),
            jax.ShapeDtypeStruct((4 * SELPAD,), jnp.float32),
            jax.ShapeDtypeStruct((4 * SELPAD,), jnp.int32),
            jax.ShapeDtypeStruct((4 * SELPAD,), jnp.float32),
            jax.ShapeDtypeStruct((4 * L,), jnp.int32),
        ],
        scratch_types=[
            pltpu.VMEM((NPAD,), jnp.float32),
            pltpu.VMEM((NPAD,), jnp.float32),
            pltpu.VMEM((NPAD,), jnp.float32),
            pltpu.VMEM((NPAD,), jnp.float32),
            pltpu.VMEM((NPAD,), jnp.float32),
            pltpu.VMEM((NPAD,), jnp.int32),
            pltpu.VMEM((NCHUNK,), jnp.float32),
            pltpu.VMEM((NM1V,), jnp.float32),
            pltpu.VMEM((SELPAD,), jnp.float32),
            pltpu.VMEM((SELPAD,), jnp.float32),
            pltpu.VMEM((SELPAD,), jnp.float32),
            pltpu.VMEM((SELPAD,), jnp.float32),
            pltpu.VMEM((SELPAD,), jnp.float32),
            pltpu.VMEM((SELPAD,), jnp.int32),
            pltpu.VMEM((SELPAD,), jnp.float32),
            pltpu.VMEM((L,), jnp.int32),
        ],
        compiler_params=pltpu.CompilerParams(needs_layout_passes=False),
    )
    def k(f_hbm, c_hbm, ox1_hbm, oy1_hbm, ox2_hbm, oy2_hbm, oc_hbm, os_hbm,
          on_hbm,
          s_v, x1_v, y1_v, x2_v, y2_v, cl_v, m1_v, m2_v,
          sx1_v, sy1_v, sx2_v, sy2_v, sar_v,
          oc_v, os_v, on_v):
        wid = lax.axis_index("c") * 16 + lax.axis_index("s")

        @pl.when(wid < 4)
        def _():
            iota16 = lax.iota(jnp.int32, 16)
            zeros16 = jnp.zeros((L,), jnp.float32)
            izeros16 = jnp.zeros((L,), jnp.int32)

            base = wid * 5 * NPAD
            pltpu.sync_copy(f_hbm.at[pl.ds(base, NPAD)], s_v)
            pltpu.sync_copy(f_hbm.at[pl.ds(base + NPAD, NPAD)], x1_v)
            pltpu.sync_copy(f_hbm.at[pl.ds(base + 2 * NPAD, NPAD)], y1_v)
            pltpu.sync_copy(f_hbm.at[pl.ds(base + 3 * NPAD, NPAD)], x2_v)
            pltpu.sync_copy(f_hbm.at[pl.ds(base + 4 * NPAD, NPAD)], y2_v)
            pltpu.sync_copy(c_hbm.at[pl.ds(wid * NPAD, NPAD)], cl_v)

            for q in range(SELPAD // L):
                oc_v[pl.ds(q * L, L)] = izeros16
                os_v[pl.ds(q * L, L)] = zeros16
                sx1_v[pl.ds(q * L, L)] = zeros16
                sy1_v[pl.ds(q * L, L)] = zeros16
                sx2_v[pl.ds(q * L, L)] = zeros16
                sy2_v[pl.ds(q * L, L)] = zeros16
                sar_v[pl.ds(q * L, L)] = zeros16
            on_v[pl.ds(0, L)] = izeros16

            def m1_body(r, carry):
                w = jnp.full((L,), NEG_INF, jnp.float32)
                for j in range(L):
                    cj = jnp.max(s_v[pl.ds(r * 256 + j * L, L)], axis=0)
                    w = jnp.where(iota16 == j, cj, w)
                m1_v[pl.ds(r * L, L)] = w
                return carry

            lax.fori_loop(0, NM1V, m1_body, jnp.int32(0))

            for i in range(NM2V):
                w = jnp.full((L,), NEG_INF, jnp.float32)
                for j in range(L):
                    cj = jnp.max(m1_v[pl.ds(i * 256 + j * L, L)], axis=0)
                    w = jnp.where(iota16 == j, cj, w)
                m2_v[pl.ds(i * L, L)] = w

            big = jnp.int32(NPAD)

            def global_argmax():
                g = m2_v[pl.ds(0, L)]
                for q in range(1, NM2V):
                    g = jnp.maximum(g, m2_v[pl.ds(q * L, L)])
                gmax = jnp.max(g, axis=0)
                pos = jnp.full((L,), NPAD, jnp.int32)
                for q in range(NM2V):
                    vv = m2_v[pl.ds(q * L, L)]
                    pos = jnp.minimum(
                        pos, jnp.where(vv == gmax, q * L + iota16, big)
                    )
                p1 = jnp.min(pos, axis=0)
                m1c = m1_v[pl.ds(p1 * L, L)]
                ch = jnp.min(
                    jnp.where(m1c == gmax, p1 * L + iota16, big), axis=0
                )
                sch = s_v[pl.ds(ch * L, L)]
                idx = jnp.min(
                    jnp.where(sch == gmax, ch * L + iota16, big), axis=0
                )
                return gmax, idx

            def extract_f(ref, idx, lane):
                v = ref[pl.ds(idx - lane, L)]
                return jnp.max(jnp.where(iota16 == lane, v, NEG_INF), axis=0)

            def extract_i(ref, idx, lane):
                v = ref[pl.ds(idx - lane, L)]
                return jnp.max(jnp.where(iota16 == lane, v, 0), axis=0)

            def append(ref, val, qn, ln):
                v = ref[pl.ds(qn * L, L)]
                ref[pl.ds(qn * L, L)] = jnp.where(iota16 == ln, val, v)

            def cond(state):
                n, gmax, _idx = state
                return (n < MAX_OUT) & (gmax != NEG_INF)

            def body(state):
                n, gmax, idx = state
                lane = idx % L
                ch = idx // L
                bx1 = extract_f(x1_v, idx, lane)
                by1 = extract_f(y1_v, idx, lane)
                bx2 = extract_f(x2_v, idx, lane)
                by2 = extract_f(y2_v, idx, lane)
                barea = (bx2 - bx1) * (by2 - by1)

                sup = jnp.zeros((L,), jnp.int32)
                for q in range(SELPAD // L):
                    qx1 = sx1_v[pl.ds(q * L, L)]
                    qy1 = sy1_v[pl.ds(q * L, L)]
                    qx2 = sx2_v[pl.ds(q * L, L)]
                    qy2 = sy2_v[pl.ds(q * L, L)]
                    qar = sar_v[pl.ds(q * L, L)]
                    xx1 = jnp.maximum(bx1, qx1)
                    yy1 = jnp.maximum(by1, qy1)
                    xx2 = jnp.minimum(bx2, qx2)
                    yy2 = jnp.minimum(by2, qy2)
                    inter = jnp.maximum(xx2 - xx1, 0.0) * jnp.maximum(
                        yy2 - yy1, 0.0
                    )
                    iou = inter / (barea + qar - inter + 1e-9)
                    hit = (q * L + iota16 < n) & (iou > IOU_THR)
                    sup = sup | hit.astype(jnp.int32)
                keep = jnp.max(sup, axis=0) == 0

                sch = s_v[pl.ds(ch * L, L)]
                sch = jnp.where(iota16 == lane, NEG_INF, sch)
                s_v[pl.ds(ch * L, L)] = sch
                cmax = jnp.max(sch, axis=0)
                q1 = ch // L
                l1 = ch % L
                m1c = m1_v[pl.ds(q1 * L, L)]
                m1c = jnp.where(iota16 == l1, cmax, m1c)
                m1_v[pl.ds(q1 * L, L)] = m1c
                nm1 = jnp.max(m1c, axis=0)
                q2 = q1 // L
                l2 = q1 % L
                m2c = m2_v[pl.ds(q2 * L, L)]
                m2c = jnp.where(iota16 == l2, nm1, m2c)
                m2_v[pl.ds(q2 * L, L)] = m2c

                @pl.when(keep)
                def _():
                    bcls = extract_i(cl_v, idx, lane)
                    qn = n // L
                    ln = n % L
                    append(sx1_v, bx1, qn, ln)
                    append(sy1_v, by1, qn, ln)
                    append(sx2_v, bx2, qn, ln)
                    append(sy2_v, by2, qn, ln)
                    append(sar_v, barea, qn, ln)
                    append(oc_v, bcls, qn, ln)
                    append(os_v, gmax, qn, ln)

                n = n + keep.astype(jnp.int32)
                gmax, idx = global_argmax()
                return n, gmax, idx

            gmax0, idx0 = global_argmax()
            state = lax.while_loop(cond, body, (jnp.int32(0), gmax0, idx0))
            nfin = state[0]
            onv = on_v[pl.ds(0, L)]
            on_v[pl.ds(0, L)] = jnp.where(iota16 == 0, nfin, onv)

            pltpu.sync_copy(sx1_v, ox1_hbm.at[pl.ds(wid * SELPAD, SELPAD)])
            pltpu.sync_copy(sy1_v, oy1_hbm.at[pl.ds(wid * SELPAD, SELPAD)])
            pltpu.sync_copy(sx2_v, ox2_hbm.at[pl.ds(wid * SELPAD, SELPAD)])
            pltpu.sync_copy(sy2_v, oy2_hbm.at[pl.ds(wid * SELPAD, SELPAD)])
            pltpu.sync_copy(oc_v, oc_hbm.at[pl.ds(wid * SELPAD, SELPAD)])
            pltpu.sync_copy(os_v, os_hbm.at[pl.ds(wid * SELPAD, SELPAD)])
            pltpu.sync_copy(on_v, on_hbm.at[pl.ds(wid * L, L)])

    return k(f2, c2)


@jax.jit
def _run(lt, db):
    f, c = _prep(lt, db)
    f2 = f.reshape(4 * 5 * NPAD)
    c2 = c.reshape(4 * NPAD)
    ox1, oy1, ox2, oy2, oc, os_, on = _sc_nms(f2, c2)
    mask = jnp.arange(MAX_OUT)[None, :] < on.reshape(4, L)[:, :1]
    det_boxes = jnp.stack(
        [
            jnp.where(mask, ox1.reshape(4, SELPAD)[:, :MAX_OUT], 0.0),
            jnp.where(mask, oy1.reshape(4, SELPAD)[:, :MAX_OUT], 0.0),
            jnp.where(mask, ox2.reshape(4, SELPAD)[:, :MAX_OUT], 0.0),
            jnp.where(mask, oy2.reshape(4, SELPAD)[:, :MAX_OUT], 0.0),
        ],
        axis=-1,
    )
    return (
        det_boxes,
        oc.reshape(4, SELPAD)[:, :MAX_OUT],
        os_.reshape(4, SELPAD)[:, :MAX_OUT],
        on.reshape(4, L)[:, 0],
    )


def kernel(logits, default_boxes):
    b, n, c = logits.shape
    lt = jnp.transpose(logits, (0, 2, 1))
    lt = jnp.pad(lt, ((0, 0), (0, 0), (0, NPAD - n))).reshape(b, c, ROWS, LANES)
    db = jnp.pad(default_boxes.T, ((0, 0), (0, NPAD - n))).reshape(4, ROWS, LANES)
    return _run(lt, db)
